# baseline (speedup 1.0000x reference)
import functools

import jax
import jax.numpy as jnp
from jax import lax
from jax.experimental import pallas as pl
from jax.experimental.pallas import tpu as pltpu
from jax.experimental.pallas import tpu_sc as plsc

_INFO = plsc.get_sparse_core_info()
_NC = _INFO.num_cores
_NS = _INFO.num_subcores
_NW = _NC * _NS

_CHUNK = 512
_BLK = _CHUNK // 128


@functools.partial(jax.jit, static_argnums=(2, 3, 4))
def _sc_gather(xt, table, bsz, h, d):
    n = bsz * h
    nblk = bsz // 128
    sb_per_h = bsz // _CHUNK
    nchunks = n // _CHUNK
    per_w = nchunks // _NW
    dg = d // 8
    assert bsz % _CHUNK == 0 and nchunks % _NW == 0 and d % 8 == 0
    assert per_w >= 4
    mesh = plsc.VectorSubcoreMesh(core_axis_name="c", subcore_axis_name="s")

    @functools.partial(
        pl.kernel,
        mesh=mesh,
        out_type=jax.ShapeDtypeStruct((h, dg, nblk, 8, 128), jnp.float32),
        scratch_types=(
            [pltpu.VMEM((_CHUNK,), jnp.int32) for _ in range(2)]
            + [pltpu.VMEM((_CHUNK, d), jnp.float32) for _ in range(2)]
            + [pltpu.VMEM((_BLK, d, 128), jnp.float32) for _ in range(2)]
            + [pltpu.SemaphoreType.DMA for _ in range(6)]
        ),
        compiler_params=pltpu.CompilerParams(use_tc_tiling_on_sc=False,
                                             needs_layout_passes=False),
    )
    def k(xt_hbm, tab_hbm, out_hbm, *scratch):
        idx_v = scratch[0:2]
        rows_v = scratch[2:4]
        t_v = scratch[4:6]
        si = scratch[6:8]
        sg = scratch[8:10]
        so = scratch[10:12]

        wid = lax.axis_index("s") * _NC + lax.axis_index("c")
        c0 = wid * per_w
        lane = lax.iota(jnp.int32, 16)

        def chunk_off(i):
            c = c0 + i
            hh = c // sb_per_h
            sb = c - hh * sb_per_h
            return hh, sb

        def idx_copy(i, p):
            hh, sb = chunk_off(i)
            return pltpu.make_async_copy(
                xt_hbm.at[pl.ds(hh * bsz + sb * _CHUNK, _CHUNK)],
                idx_v[p], si[p])

        def gather_copy(p):
            return pltpu.make_async_copy(tab_hbm.at[idx_v[p]], rows_v[p],
                                         sg[p])

        def out_copy(i, p, g):
            hh, sb = chunk_off(i)
            return pltpu.make_async_copy(
                t_v[p].at[:, pl.ds(g * 8, 8), :],
                out_hbm.at[hh, g, pl.ds(sb * _BLK, _BLK)], so[p])

        def transpose(p):
            rows = rows_v[p]
            dst = t_v[p]
            for bb in range(_BLK):
                row_idx = [lane + (bb * 128 + l * 16) for l in range(8)]

                def col(c, carry):
                    cc = jnp.full((16,), c, jnp.int32)
                    for l in range(8):
                        vals = plsc.load_gather(rows, [row_idx[l], cc])
                        dst[bb, c, pl.ds(l * 16, 16)] = vals
                    return carry

                lax.fori_loop(0, d, col, 0)

        def body(i, p, prefetch, start_next, wait_out):
            gather_copy(p).wait()
            if prefetch:
                idx_copy(i + 2, p).start()
            if start_next:
                idx_copy(0, 1 - p).wait()
                gather_copy(1 - p).start()
            if wait_out:
                for g in range(dg):
                    out_copy(0, p, g).wait()
            transpose(p)
            for g in range(dg):
                out_copy(i, p, g).start()

        idx_copy(0, 0).start()
        idx_copy(1, 1).start()
        idx_copy(0, 0).wait()
        gather_copy(0).start()

        body(0, 0, prefetch=True, start_next=True, wait_out=False)
        body(1, 1, prefetch=True, start_next=True, wait_out=False)

        def loop(j, carry):
            i = 2 + 2 * j
            body(i, 0, prefetch=True, start_next=True, wait_out=True)
            body(i + 1, 1, prefetch=True, start_next=True, wait_out=True)
            return carry

        lax.fori_loop(0, (per_w - 4) // 2, loop, 0)

        body(per_w - 2, 0, prefetch=False, start_next=True, wait_out=True)
        body(per_w - 1, 1, prefetch=False, start_next=False, wait_out=True)

        for p in range(2):
            for g in range(dg):
                out_copy(0, p, g).wait()

    return k(xt, table)


def kernel(x, table):
    b, h = x.shape
    v, d = table.shape
    xt = x.T.reshape(b * h)
    out5 = _sc_gather(xt, table, b, h, d)
    return out5.transpose(2, 4, 0, 1, 3).reshape(b, h, d)


# speedup vs baseline: 1.1824x; 1.1824x over previous
import functools

import jax
import jax.numpy as jnp
from jax import lax
from jax.experimental import pallas as pl
from jax.experimental.pallas import tpu as pltpu
from jax.experimental.pallas import tpu_sc as plsc

_INFO = plsc.get_sparse_core_info()
_NC = _INFO.num_cores
_NS = _INFO.num_subcores
_NW = _NC * _NS

_CHUNK = 512
_BLK = _CHUNK // 128


@functools.partial(jax.jit, static_argnums=(2, 3, 4))
def _sc_gather(xt, table, bsz, h, d):
    n = bsz * h
    nblk = bsz // 128
    sb_per_h = bsz // _CHUNK
    nchunks = n // _CHUNK
    per_w = nchunks // _NW
    dg = d // 8
    assert bsz % _CHUNK == 0 and nchunks % _NW == 0 and d % 8 == 0
    assert per_w >= 4
    mesh = plsc.VectorSubcoreMesh(core_axis_name="c", subcore_axis_name="s")

    @functools.partial(
        pl.kernel,
        mesh=mesh,
        out_type=jax.ShapeDtypeStruct((h, dg, nblk, 1024), jnp.float32),
        scratch_types=(
            [pltpu.VMEM((_CHUNK,), jnp.int32) for _ in range(2)]
            + [pltpu.VMEM((_CHUNK, d), jnp.float32) for _ in range(2)]
            + [pltpu.VMEM((_BLK * d * 128,), jnp.float32) for _ in range(2)]
            + [pltpu.SemaphoreType.DMA for _ in range(6)]
        ),
        compiler_params=pltpu.CompilerParams(use_tc_tiling_on_sc=False,
                                             needs_layout_passes=False),
    )
    def k(xt_hbm, tab_hbm, out_hbm, *scratch):
        idx_v = scratch[0:2]
        rows_v = scratch[2:4]
        t_v = scratch[4:6]
        si = scratch[6:8]
        sg = scratch[8:10]
        so = scratch[10:12]

        wid = lax.axis_index("s") * _NC + lax.axis_index("c")
        c0 = wid * per_w
        lane = lax.iota(jnp.int32, 16)

        def chunk_off(i):
            c = c0 + i
            hh = c // sb_per_h
            sb = c - hh * sb_per_h
            return hh, sb

        def idx_copy(i, p):
            hh, sb = chunk_off(i)
            return pltpu.make_async_copy(
                xt_hbm.at[pl.ds(hh * bsz + sb * _CHUNK, _CHUNK)],
                idx_v[p], si[p])

        def gather_copy(p):
            return pltpu.make_async_copy(tab_hbm.at[idx_v[p]], rows_v[p],
                                         sg[p])

        def out_copy(i, p, g, bb):
            hh, sb = chunk_off(i)
            return pltpu.make_async_copy(
                t_v[p].at[pl.ds(bb * d * 128 + g * 1024, 1024)],
                out_hbm.at[hh, g, sb * _BLK + bb], so[p])

        def transpose(p):
            rows = rows_v[p]
            dst = t_v[p]
            for bb in range(_BLK):
                base = [(lane + cg * 16) * 128 + bb * (d * 128)
                        for cg in range(d // 16)]

                def jbody(j, carry):
                    row = bb * 128 + j
                    for cg in range(d // 16):
                        v = rows[row, pl.ds(cg * 16, 16)]
                        plsc.store_scatter(dst, [base[cg] + j], v)
                    return carry

                lax.fori_loop(0, 128, jbody, 0, unroll=8)

        def body(i, p, prefetch, start_next, wait_out):
            gather_copy(p).wait()
            if prefetch:
                idx_copy(i + 2, p).start()
            if start_next:
                idx_copy(0, 1 - p).wait()
                gather_copy(1 - p).start()
            if wait_out:
                for g in range(dg):
                    for bb in range(_BLK):
                        out_copy(0, p, g, bb).wait()
            transpose(p)
            for g in range(dg):
                for bb in range(_BLK):
                    out_copy(i, p, g, bb).start()

        idx_copy(0, 0).start()
        idx_copy(1, 1).start()
        idx_copy(0, 0).wait()
        gather_copy(0).start()

        body(0, 0, prefetch=True, start_next=True, wait_out=False)
        body(1, 1, prefetch=True, start_next=True, wait_out=False)

        def loop(j, carry):
            i = 2 + 2 * j
            body(i, 0, prefetch=True, start_next=True, wait_out=True)
            body(i + 1, 1, prefetch=True, start_next=True, wait_out=True)
            return carry

        lax.fori_loop(0, (per_w - 4) // 2, loop, 0)

        body(per_w - 2, 0, prefetch=False, start_next=True, wait_out=True)
        body(per_w - 1, 1, prefetch=False, start_next=False, wait_out=True)

        for p in range(2):
            for g in range(dg):
                for bb in range(_BLK):
                    out_copy(0, p, g, bb).wait()

    return k(xt, table)


def kernel(x, table):
    b, h = x.shape
    v, d = table.shape
    xt = x.T.reshape(b * h)
    out5 = _sc_gather(xt, table, b, h, d)
    out5 = out5.reshape(h, d // 8, b // 128, 8, 128)
    return out5.transpose(2, 4, 0, 1, 3).reshape(b, h, d)
